# baseline (speedup 1.0000x reference)
import functools

import jax
import jax.numpy as jnp
from jax import lax
from jax.experimental import pallas as pl
from jax.experimental.pallas import tpu as pltpu
from jax.experimental.pallas import tpu_sc as plsc

N = 100000
D = 128
S = 256
U = 100
SR = 16
C = 160
NCH = N // C
GPC = C // 16
NW = 32
NBUF = 3
MAXCH = (NCH + NW - 1) // NW
MAXSLOT = -(-MAXCH // NBUF) * NBUF

_mesh = plsc.VectorSubcoreMesh(core_axis_name="c", subcore_axis_name="s")
_params = pltpu.CompilerParams(needs_layout_passes=False)


@functools.partial(
    pl.kernel,
    out_type=(jax.ShapeDtypeStruct((2, S, D), jnp.float32),
              jax.ShapeDtypeStruct((2, SR, D), jnp.float32)),
    mesh=_mesh,
    compiler_params=_params,
    scratch_types=[
        pltpu.VMEM((NBUF * C * D,), jnp.float32),
        pltpu.VMEM((NBUF * C,), jnp.int32),
        pltpu.VMEM((NBUF * C,), jnp.int32),
        pltpu.VMEM((D,), jnp.float32),
        pltpu.VMEM((D,), jnp.float32),
        pltpu.VMEM((D,), jnp.float32),
        pltpu.VMEM((S, D), jnp.float32),
        pltpu.VMEM((SR, D), jnp.float32),
        pltpu.VMEM((GPC * S,), jnp.float32),
        pltpu.VMEM_SHARED((S, D), jnp.float32),
        pltpu.VMEM_SHARED((SR, D), jnp.float32),
        pltpu.VMEM((S // 2,), jnp.int32),
        pltpu.VMEM((S // 2,), jnp.int32),
        pltpu.VMEM((SR,), jnp.int32),
        pltpu.SemaphoreType.DMA((NBUF,)),
        pltpu.SemaphoreType.DMA((NBUF,)),
        pltpu.SemaphoreType.DMA((NBUF,)),
    ],
)
def _phase1(h_hbm, seg_hbm, typ_hbm, w_hbm, a_hbm, c_hbm,
            part_hbm, partsc_hbm,
            hbuf, segbuf, typbuf, wbuf, abuf, cbuf, acc, accsc, dsc,
            shared, sharedsc, idxa, idxb, idxc, hsem, ssem, tsem):
    sid = lax.axis_index("s")
    cid = lax.axis_index("c")
    wid = sid * 2 + cid
    pltpu.sync_copy(w_hbm, wbuf)
    pltpu.sync_copy(a_hbm, abuf)
    pltpu.sync_copy(c_hbm, cbuf)

    zeros16 = jnp.zeros((16,), jnp.float32)
    iota16 = lax.iota(jnp.int32, 16)

    def zero_body(i, carry):
        for j in range(D // 16):
            acc[i, pl.ds(16 * j, 16)] = zeros16
        return carry

    lax.fori_loop(0, S, zero_body, 0)
    for i in range(SR):
        for j in range(D // 16):
            accsc[i, pl.ds(16 * j, 16)] = zeros16
    for k in range(8):
        idxa[pl.ds(16 * k, 16)] = iota16 + 16 * k
        idxb[pl.ds(16 * k, 16)] = iota16 + (128 + 16 * k)
    idxc[...] = iota16

    @pl.when(sid == 0)
    def _():
        pltpu.sync_copy(acc, shared)
        pltpu.sync_copy(accsc, sharedsc)

    wvecs = [wbuf[pl.ds(16 * j, 16)] for j in range(8)]
    dsc_idx = iota16 * 16

    nchunks = (NCH - wid + NW - 1) // NW

    def _issue(b, i):
        chunk = wid + i * NW
        off = chunk * C
        pltpu.async_copy(seg_hbm.at[pl.ds(off, C)],
                         segbuf.at[pl.ds(b * C, C)], ssem.at[b])
        pltpu.async_copy(typ_hbm.at[pl.ds(off, C)],
                         typbuf.at[pl.ds(b * C, C)], tsem.at[b])
        pltpu.async_copy(h_hbm.at[pl.ds(chunk * (C * D), C * D)],
                         hbuf.at[pl.ds(b * C * D, C * D)], hsem.at[b])

    def _wait(b):
        pltpu.make_async_copy(seg_hbm.at[pl.ds(0, C)],
                              segbuf.at[pl.ds(b * C, C)], ssem.at[b]).wait()
        pltpu.make_async_copy(typ_hbm.at[pl.ds(0, C)],
                              typbuf.at[pl.ds(b * C, C)], tsem.at[b]).wait()
        pltpu.make_async_copy(h_hbm.at[pl.ds(0, C * D)],
                              hbuf.at[pl.ds(b * C * D, C * D)], hsem.at[b]).wait()

    def _process(b):
        hb0 = b * C * D
        sb0 = b * C

        @plsc.parallel_loop(0, C, 1, unroll=1)
        def row_body(r):
            rb = hb0 + r * D
            seg_b = plsc.load_gather(
                segbuf, [jnp.full((16,), sb0 + r, jnp.int32)])
            prods = []
            for j in range(8):
                hv = hbuf[pl.ds(rb + 16 * j, 16)]
                plsc.addupdate_scatter(acc, [seg_b, iota16 + (16 * j)], hv)
                prods.append(hv * wvecs[j])
            m = ((prods[0] + prods[1]) + (prods[2] + prods[3])) + \
                ((prods[4] + prods[5]) + (prods[6] + prods[7]))
            base_s = ((r >> 4) << 8) | (r & 15)
            plsc.store_scatter(dsc, [dsc_idx + base_s], m)

        @plsc.parallel_loop(0, GPC, 1)
        def group_body(g):
            seg_vec = segbuf[pl.ds(sb0 + g * 16, 16)]
            typ_vec = typbuf[pl.ds(sb0 + g * 16, 16)]
            a_g = plsc.load_gather(abuf, [typ_vec])
            c_g = plsc.load_gather(cbuf, [typ_vec])
            db = g * S
            vs = [dsc[pl.ds(db + 16 * l, 16)] for l in range(16)]
            while len(vs) > 1:
                vs = [vs[i] + vs[i + 1] for i in range(0, len(vs), 2)]
            f_vec = a_g * vs[0] + c_g
            plsc.addupdate_scatter(
                accsc, [seg_vec >> 4, seg_vec & 15], f_vec)

    for p in range(NBUF - 1):
        @pl.when(p < nchunks)
        def _(p=p):
            _issue(p, p)

    def outer(k2, carry):
        for b in range(NBUF):
            i = k2 * NBUF + b

            @pl.when(i < nchunks)
            def _():
                _wait(b)

                @pl.when(i + (NBUF - 1) < nchunks)
                def _():
                    _issue((b + NBUF - 1) % NBUF, i + (NBUF - 1))

                _process(b)
        return carry

    lax.fori_loop(0, MAXSLOT // NBUF, outer, 0)
    plsc.subcore_barrier()
    pltpu.sync_copy(acc.at[pl.ds(0, S // 2)], shared.at[idxa], add=True)
    pltpu.sync_copy(acc.at[pl.ds(S // 2, S // 2)], shared.at[idxb], add=True)
    pltpu.sync_copy(accsc, sharedsc.at[idxc], add=True)
    plsc.subcore_barrier()
    rows = S // 16
    pltpu.sync_copy(shared.at[pl.ds(sid * rows, rows)],
                    part_hbm.at[cid, pl.ds(sid * rows, rows)])

    @pl.when(sid == 0)
    def _():
        pltpu.sync_copy(sharedsc, partsc_hbm.at[cid])


def _phase2_body(p1_ref, p2_ref, o1_ref, o2_ref):
    i = pl.program_id(0)

    @pl.when(i == 0)
    def _():
        o1_ref[...] = p1_ref[0]
        o2_ref[...] = p2_ref[0]

    @pl.when(i > 0)
    def _():
        o1_ref[...] += p1_ref[0]
        o2_ref[...] += p2_ref[0]


_phase2 = pl.pallas_call(
    _phase2_body,
    grid=(2,),
    in_specs=[pl.BlockSpec((1, 1, S * D), lambda i: (i, 0, 0)),
              pl.BlockSpec((1, 1, SR * D), lambda i: (i, 0, 0))],
    out_specs=[pl.BlockSpec((1, S * D), lambda i: (0, 0)),
               pl.BlockSpec((1, SR * D), lambda i: (0, 0))],
    out_shape=[jax.ShapeDtypeStruct((1, S * D), jnp.float32),
               jax.ShapeDtypeStruct((1, SR * D), jnp.float32)],
)


def kernel(h, node_feat_discrete, segment_ids, W, b, scale_table, bias_table,
           mean, std):
    h = h.astype(jnp.float32)
    seg = segment_ids.astype(jnp.int32)
    typ = node_feat_discrete.astype(jnp.int32)
    std0 = std.astype(jnp.float32)[0]
    a_small = std0 * scale_table.astype(jnp.float32)[:, 0]
    c_small = (a_small * b.astype(jnp.float32)[0]
               + std0 * bias_table.astype(jnp.float32)[:, 0]
               + mean.astype(jnp.float32)[0])
    a_pad = jnp.zeros((D,), jnp.float32).at[:U].set(a_small)
    c_pad = jnp.zeros((D,), jnp.float32).at[:U].set(c_small)
    partials, partsc = _phase1(h.reshape(-1), seg, typ,
                               W.astype(jnp.float32).reshape(-1),
                               a_pad, c_pad)
    gf_flat, sc_flat = _phase2(partials.reshape(2, 1, S * D),
                               partsc.reshape(2, 1, SR * D))
    graph_feat = gf_flat.reshape(S, D)
    node_score = sc_flat.reshape(SR, D)[:, :16].reshape(S, 1)
    return (graph_feat, node_score)


# speedup vs baseline: 1.0108x; 1.0108x over previous
import functools

import jax
import jax.numpy as jnp
from jax import lax
from jax.experimental import pallas as pl
from jax.experimental.pallas import tpu as pltpu
from jax.experimental.pallas import tpu_sc as plsc

N = 100000
D = 128
S = 256
U = 100
SR = 16
C = 160
NCH = N // C
GPC = C // 16
NW = 32
NBUF = 2
MAXCH = (NCH + NW - 1) // NW
MAXSLOT = -(-MAXCH // NBUF) * NBUF

_mesh = plsc.VectorSubcoreMesh(core_axis_name="c", subcore_axis_name="s")
_params = pltpu.CompilerParams(needs_layout_passes=False)


@functools.partial(
    pl.kernel,
    out_type=(jax.ShapeDtypeStruct((2, S, D), jnp.float32),
              jax.ShapeDtypeStruct((2, SR, D), jnp.float32)),
    mesh=_mesh,
    compiler_params=_params,
    scratch_types=[
        pltpu.VMEM((NBUF * C * D,), jnp.float32),
        pltpu.VMEM((NBUF * C,), jnp.int32),
        pltpu.VMEM((NBUF * C,), jnp.int32),
        pltpu.VMEM((D,), jnp.float32),
        pltpu.VMEM((D,), jnp.float32),
        pltpu.VMEM((D,), jnp.float32),
        pltpu.VMEM((S, D), jnp.float32),
        pltpu.VMEM((SR, D), jnp.float32),
        pltpu.VMEM((GPC * S,), jnp.float32),
        pltpu.VMEM_SHARED((S, D), jnp.float32),
        pltpu.VMEM_SHARED((SR, D), jnp.float32),
        pltpu.VMEM((S // 2,), jnp.int32),
        pltpu.VMEM((S // 2,), jnp.int32),
        pltpu.VMEM((SR,), jnp.int32),
        pltpu.SemaphoreType.DMA((NBUF,)),
        pltpu.SemaphoreType.DMA((NBUF,)),
        pltpu.SemaphoreType.DMA((NBUF,)),
    ],
)
def _phase1(h_hbm, seg_hbm, typ_hbm, w_hbm, a_hbm, c_hbm,
            part_hbm, partsc_hbm,
            hbuf, segbuf, typbuf, wbuf, abuf, cbuf, acc, accsc, dsc,
            shared, sharedsc, idxa, idxb, idxc, hsem, ssem, tsem):
    sid = lax.axis_index("s")
    cid = lax.axis_index("c")
    wid = sid * 2 + cid
    pltpu.sync_copy(w_hbm, wbuf)
    pltpu.sync_copy(a_hbm, abuf)
    pltpu.sync_copy(c_hbm, cbuf)

    zeros16 = jnp.zeros((16,), jnp.float32)
    iota16 = lax.iota(jnp.int32, 16)

    def zero_body(i, carry):
        for j in range(D // 16):
            acc[i, pl.ds(16 * j, 16)] = zeros16
        return carry

    lax.fori_loop(0, S, zero_body, 0)
    for i in range(SR):
        for j in range(D // 16):
            accsc[i, pl.ds(16 * j, 16)] = zeros16
    for k in range(8):
        idxa[pl.ds(16 * k, 16)] = iota16 + 16 * k
        idxb[pl.ds(16 * k, 16)] = iota16 + (128 + 16 * k)
    idxc[...] = iota16

    @pl.when(sid == 0)
    def _():
        pltpu.sync_copy(acc, shared)
        pltpu.sync_copy(accsc, sharedsc)

    wvecs = [wbuf[pl.ds(16 * j, 16)] for j in range(8)]
    dsc_idx = iota16 * 16

    nchunks = (NCH - wid + NW - 1) // NW

    def _issue(b, i):
        chunk = wid + i * NW
        off = chunk * C
        pltpu.async_copy(seg_hbm.at[pl.ds(off, C)],
                         segbuf.at[pl.ds(b * C, C)], ssem.at[b])
        pltpu.async_copy(typ_hbm.at[pl.ds(off, C)],
                         typbuf.at[pl.ds(b * C, C)], tsem.at[b])
        pltpu.async_copy(h_hbm.at[pl.ds(chunk * (C * D), C * D)],
                         hbuf.at[pl.ds(b * C * D, C * D)], hsem.at[b])

    def _wait(b):
        pltpu.make_async_copy(seg_hbm.at[pl.ds(0, C)],
                              segbuf.at[pl.ds(b * C, C)], ssem.at[b]).wait()
        pltpu.make_async_copy(typ_hbm.at[pl.ds(0, C)],
                              typbuf.at[pl.ds(b * C, C)], tsem.at[b]).wait()
        pltpu.make_async_copy(h_hbm.at[pl.ds(0, C * D)],
                              hbuf.at[pl.ds(b * C * D, C * D)], hsem.at[b]).wait()

    def _process(b):
        hb0 = b * C * D
        sb0 = b * C

        @plsc.parallel_loop(0, C, 1, unroll=1)
        def row_body(r):
            rb = hb0 + r * D
            seg_b = plsc.load_gather(
                segbuf, [jnp.full((16,), sb0 + r, jnp.int32)])
            prods = []
            for j in range(8):
                hv = hbuf[pl.ds(rb + 16 * j, 16)]
                plsc.addupdate_scatter(acc, [seg_b, iota16 + (16 * j)], hv)
                prods.append(hv * wvecs[j])
            m = ((prods[0] + prods[1]) + (prods[2] + prods[3])) + \
                ((prods[4] + prods[5]) + (prods[6] + prods[7]))
            base_s = ((r >> 4) << 8) | (r & 15)
            plsc.store_scatter(dsc, [dsc_idx + base_s], m)

        @plsc.parallel_loop(0, GPC, 1)
        def group_body(g):
            seg_vec = segbuf[pl.ds(sb0 + g * 16, 16)]
            typ_vec = typbuf[pl.ds(sb0 + g * 16, 16)]
            a_g = plsc.load_gather(abuf, [typ_vec])
            c_g = plsc.load_gather(cbuf, [typ_vec])
            db = g * S
            vs = [dsc[pl.ds(db + 16 * l, 16)] for l in range(16)]
            while len(vs) > 1:
                vs = [vs[i] + vs[i + 1] for i in range(0, len(vs), 2)]
            f_vec = a_g * vs[0] + c_g
            plsc.addupdate_scatter(
                accsc, [seg_vec >> 4, seg_vec & 15], f_vec)

    for p in range(NBUF - 1):
        @pl.when(p < nchunks)
        def _(p=p):
            _issue(p, p)

    def outer(k2, carry):
        for b in range(NBUF):
            i = k2 * NBUF + b

            @pl.when(i < nchunks)
            def _():
                _wait(b)

                @pl.when(i + (NBUF - 1) < nchunks)
                def _():
                    _issue((b + NBUF - 1) % NBUF, i + (NBUF - 1))

                _process(b)
        return carry

    lax.fori_loop(0, MAXSLOT // NBUF, outer, 0)
    plsc.subcore_barrier()
    pltpu.sync_copy(acc.at[pl.ds(0, S // 2)], shared.at[idxa], add=True)
    pltpu.sync_copy(acc.at[pl.ds(S // 2, S // 2)], shared.at[idxb], add=True)
    pltpu.sync_copy(accsc, sharedsc.at[idxc], add=True)
    plsc.subcore_barrier()
    rows = S // 16
    pltpu.sync_copy(shared.at[pl.ds(sid * rows, rows)],
                    part_hbm.at[cid, pl.ds(sid * rows, rows)])

    @pl.when(sid == 0)
    def _():
        pltpu.sync_copy(sharedsc, partsc_hbm.at[cid])


def _phase2_body(p1_ref, p2_ref, o1_ref, o2_ref):
    i = pl.program_id(0)

    @pl.when(i == 0)
    def _():
        o1_ref[...] = p1_ref[0]
        o2_ref[...] = p2_ref[0]

    @pl.when(i > 0)
    def _():
        o1_ref[...] += p1_ref[0]
        o2_ref[...] += p2_ref[0]


_phase2 = pl.pallas_call(
    _phase2_body,
    grid=(2,),
    in_specs=[pl.BlockSpec((1, 1, S * D), lambda i: (i, 0, 0)),
              pl.BlockSpec((1, 1, SR * D), lambda i: (i, 0, 0))],
    out_specs=[pl.BlockSpec((1, S * D), lambda i: (0, 0)),
               pl.BlockSpec((1, SR * D), lambda i: (0, 0))],
    out_shape=[jax.ShapeDtypeStruct((1, S * D), jnp.float32),
               jax.ShapeDtypeStruct((1, SR * D), jnp.float32)],
)


def kernel(h, node_feat_discrete, segment_ids, W, b, scale_table, bias_table,
           mean, std):
    h = h.astype(jnp.float32)
    seg = segment_ids.astype(jnp.int32)
    typ = node_feat_discrete.astype(jnp.int32)
    std0 = std.astype(jnp.float32)[0]
    a_small = std0 * scale_table.astype(jnp.float32)[:, 0]
    c_small = (a_small * b.astype(jnp.float32)[0]
               + std0 * bias_table.astype(jnp.float32)[:, 0]
               + mean.astype(jnp.float32)[0])
    a_pad = jnp.zeros((D,), jnp.float32).at[:U].set(a_small)
    c_pad = jnp.zeros((D,), jnp.float32).at[:U].set(c_small)
    partials, partsc = _phase1(h.reshape(-1), seg, typ,
                               W.astype(jnp.float32).reshape(-1),
                               a_pad, c_pad)
    gf_flat, sc_flat = _phase2(partials.reshape(2, 1, S * D),
                               partsc.reshape(2, 1, SR * D))
    graph_feat = gf_flat.reshape(S, D)
    node_score = sc_flat.reshape(SR, D)[:, :16].reshape(S, 1)
    return (graph_feat, node_score)


# speedup vs baseline: 1.0243x; 1.0134x over previous
import functools

import jax
import jax.numpy as jnp
from jax import lax
from jax.experimental import pallas as pl
from jax.experimental.pallas import tpu as pltpu
from jax.experimental.pallas import tpu_sc as plsc

N = 100000
D = 128
S = 256
U = 100
SR = 16
C = 160
NCH = N // C
GPC = C // 16
NW = 32
NBUF = 2
MAXCH = (NCH + NW - 1) // NW
MAXSLOT = -(-MAXCH // NBUF) * NBUF

_mesh = plsc.VectorSubcoreMesh(core_axis_name="c", subcore_axis_name="s")
_params = pltpu.CompilerParams(needs_layout_passes=False)


@functools.partial(
    pl.kernel,
    out_type=(jax.ShapeDtypeStruct((2, S, D), jnp.float32),
              jax.ShapeDtypeStruct((2, SR, D), jnp.float32)),
    mesh=_mesh,
    compiler_params=_params,
    scratch_types=[
        pltpu.VMEM((NBUF * C * D,), jnp.float32),
        pltpu.VMEM((NBUF * C,), jnp.int32),
        pltpu.VMEM((NBUF * C,), jnp.int32),
        pltpu.VMEM((D,), jnp.float32),
        pltpu.VMEM((D,), jnp.float32),
        pltpu.VMEM((D,), jnp.float32),
        pltpu.VMEM((S, D), jnp.float32),
        pltpu.VMEM((SR, D), jnp.float32),
        pltpu.VMEM((GPC * S,), jnp.float32),
        pltpu.VMEM_SHARED((S, D), jnp.float32),
        pltpu.VMEM_SHARED((SR, D), jnp.float32),
        pltpu.VMEM((S // 2,), jnp.int32),
        pltpu.VMEM((S // 2,), jnp.int32),
        pltpu.VMEM((SR,), jnp.int32),
        pltpu.SemaphoreType.DMA((NBUF,)),
        pltpu.SemaphoreType.DMA((NBUF,)),
        pltpu.SemaphoreType.DMA((NBUF,)),
    ],
)
def _phase1(h_hbm, seg_hbm, typ_hbm, w_hbm, a_hbm, c_hbm,
            part_hbm, partsc_hbm,
            hbuf, segbuf, typbuf, wbuf, abuf, cbuf, acc, accsc, dsc,
            shared, sharedsc, idxa, idxb, idxc, hsem, ssem, tsem):
    sid = lax.axis_index("s")
    cid = lax.axis_index("c")
    wid = sid * 2 + cid

    nchunks = (NCH - wid + NW - 1) // NW

    def _issue(b, i):
        chunk = wid + i * NW
        off = chunk * C
        pltpu.async_copy(seg_hbm.at[pl.ds(off, C)],
                         segbuf.at[pl.ds(b * C, C)], ssem.at[b])
        pltpu.async_copy(typ_hbm.at[pl.ds(off, C)],
                         typbuf.at[pl.ds(b * C, C)], tsem.at[b])
        pltpu.async_copy(h_hbm.at[pl.ds(chunk * (C * D), C * D)],
                         hbuf.at[pl.ds(b * C * D, C * D)], hsem.at[b])

    for p in range(NBUF - 1):
        @pl.when(p < nchunks)
        def _(p=p):
            _issue(p, p)

    pltpu.sync_copy(w_hbm, wbuf)
    pltpu.sync_copy(a_hbm, abuf)
    pltpu.sync_copy(c_hbm, cbuf)

    zeros16 = jnp.zeros((16,), jnp.float32)
    iota16 = lax.iota(jnp.int32, 16)

    def zero_body(i, carry):
        for j in range(D // 16):
            acc[i, pl.ds(16 * j, 16)] = zeros16
        return carry

    lax.fori_loop(0, S, zero_body, 0)
    for i in range(SR):
        for j in range(D // 16):
            accsc[i, pl.ds(16 * j, 16)] = zeros16
    for k in range(8):
        idxa[pl.ds(16 * k, 16)] = iota16 + 16 * k
        idxb[pl.ds(16 * k, 16)] = iota16 + (128 + 16 * k)
    idxc[...] = iota16

    @pl.when(sid == 0)
    def _():
        pltpu.sync_copy(acc, shared)
        pltpu.sync_copy(accsc, sharedsc)

    wvecs = [wbuf[pl.ds(16 * j, 16)] for j in range(8)]
    dsc_idx = iota16 * 16

    def _wait(b):
        pltpu.make_async_copy(seg_hbm.at[pl.ds(0, C)],
                              segbuf.at[pl.ds(b * C, C)], ssem.at[b]).wait()
        pltpu.make_async_copy(typ_hbm.at[pl.ds(0, C)],
                              typbuf.at[pl.ds(b * C, C)], tsem.at[b]).wait()
        pltpu.make_async_copy(h_hbm.at[pl.ds(0, C * D)],
                              hbuf.at[pl.ds(b * C * D, C * D)], hsem.at[b]).wait()

    def _process(b):
        hb0 = b * C * D
        sb0 = b * C

        @plsc.parallel_loop(0, C, 1, unroll=1)
        def row_body(r):
            rb = hb0 + r * D
            seg_b = plsc.load_gather(
                segbuf, [jnp.full((16,), sb0 + r, jnp.int32)])
            prods = []
            for j in range(8):
                hv = hbuf[pl.ds(rb + 16 * j, 16)]
                plsc.addupdate_scatter(acc, [seg_b, iota16 + (16 * j)], hv)
                prods.append(hv * wvecs[j])
            m = ((prods[0] + prods[1]) + (prods[2] + prods[3])) + \
                ((prods[4] + prods[5]) + (prods[6] + prods[7]))
            base_s = ((r >> 4) << 8) | (r & 15)
            plsc.store_scatter(dsc, [dsc_idx + base_s], m)

        @plsc.parallel_loop(0, GPC, 1)
        def group_body(g):
            seg_vec = segbuf[pl.ds(sb0 + g * 16, 16)]
            typ_vec = typbuf[pl.ds(sb0 + g * 16, 16)]
            a_g = plsc.load_gather(abuf, [typ_vec])
            c_g = plsc.load_gather(cbuf, [typ_vec])
            db = g * S
            vs = [dsc[pl.ds(db + 16 * l, 16)] for l in range(16)]
            while len(vs) > 1:
                vs = [vs[i] + vs[i + 1] for i in range(0, len(vs), 2)]
            f_vec = a_g * vs[0] + c_g
            plsc.addupdate_scatter(
                accsc, [seg_vec >> 4, seg_vec & 15], f_vec)

    def outer(k2, carry):
        for b in range(NBUF):
            i = k2 * NBUF + b

            @pl.when(i < nchunks)
            def _():
                _wait(b)

                @pl.when(i + (NBUF - 1) < nchunks)
                def _():
                    _issue((b + NBUF - 1) % NBUF, i + (NBUF - 1))

                _process(b)
        return carry

    lax.fori_loop(0, MAXSLOT // NBUF, outer, 0)
    plsc.subcore_barrier()
    pltpu.sync_copy(acc.at[pl.ds(0, S // 2)], shared.at[idxa], add=True)
    pltpu.sync_copy(acc.at[pl.ds(S // 2, S // 2)], shared.at[idxb], add=True)
    pltpu.sync_copy(accsc, sharedsc.at[idxc], add=True)
    plsc.subcore_barrier()
    rows = S // 16
    pltpu.sync_copy(shared.at[pl.ds(sid * rows, rows)],
                    part_hbm.at[cid, pl.ds(sid * rows, rows)])

    @pl.when(sid == 0)
    def _():
        pltpu.sync_copy(sharedsc, partsc_hbm.at[cid])


def _phase2_body(p1_ref, p2_ref, o1_ref, o2_ref):
    i = pl.program_id(0)

    @pl.when(i == 0)
    def _():
        o1_ref[...] = p1_ref[0]
        o2_ref[...] = p2_ref[0]

    @pl.when(i > 0)
    def _():
        o1_ref[...] += p1_ref[0]
        o2_ref[...] += p2_ref[0]


_phase2 = pl.pallas_call(
    _phase2_body,
    grid=(2,),
    in_specs=[pl.BlockSpec((1, 1, S * D), lambda i: (i, 0, 0)),
              pl.BlockSpec((1, 1, SR * D), lambda i: (i, 0, 0))],
    out_specs=[pl.BlockSpec((1, S * D), lambda i: (0, 0)),
               pl.BlockSpec((1, SR * D), lambda i: (0, 0))],
    out_shape=[jax.ShapeDtypeStruct((1, S * D), jnp.float32),
               jax.ShapeDtypeStruct((1, SR * D), jnp.float32)],
)


def kernel(h, node_feat_discrete, segment_ids, W, b, scale_table, bias_table,
           mean, std):
    h = h.astype(jnp.float32)
    seg = segment_ids.astype(jnp.int32)
    typ = node_feat_discrete.astype(jnp.int32)
    std0 = std.astype(jnp.float32)[0]
    a_small = std0 * scale_table.astype(jnp.float32)[:, 0]
    c_small = (a_small * b.astype(jnp.float32)[0]
               + std0 * bias_table.astype(jnp.float32)[:, 0]
               + mean.astype(jnp.float32)[0])
    a_pad = jnp.zeros((D,), jnp.float32).at[:U].set(a_small)
    c_pad = jnp.zeros((D,), jnp.float32).at[:U].set(c_small)
    partials, partsc = _phase1(h.reshape(-1), seg, typ,
                               W.astype(jnp.float32).reshape(-1),
                               a_pad, c_pad)
    gf_flat, sc_flat = _phase2(partials.reshape(2, 1, S * D),
                               partsc.reshape(2, 1, SR * D))
    graph_feat = gf_flat.reshape(S, D)
    node_score = sc_flat.reshape(SR, D)[:, :16].reshape(S, 1)
    return (graph_feat, node_score)
